# baseline (device time: 107534 ns/iter reference)
import jax
import jax.numpy as jnp
from jax import lax
from jax.experimental import pallas as pl
from jax.experimental.pallas import tpu as pltpu

N_DEV = 32


def kernel(x, w_mat):
    m_per, k = x.shape
    _, n_per = w_mat.shape
    m_total = N_DEV * m_per

    def body(x_ref, w_ref, out_ref, comm_ref, send_sems, recv_sems):
        my = lax.axis_index("i")
        left = (my - 1) % N_DEV
        right = (my + 1) % N_DEV

        barrier_sem = pltpu.get_barrier_semaphore()
        for nbr in (left, right):
            pl.semaphore_signal(
                barrier_sem, inc=1,
                device_id=(nbr,), device_id_type=pl.DeviceIdType.MESH,
            )
        pl.semaphore_wait(barrier_sem, 2)

        comm_ref[pl.ds(my * m_per, m_per), :] = x_ref[...]

        for h in range(N_DEV - 1):
            src_o = (my - h) % N_DEV
            rdma = pltpu.make_async_remote_copy(
                src_ref=comm_ref.at[pl.ds(src_o * m_per, m_per), :],
                dst_ref=comm_ref.at[pl.ds(src_o * m_per, m_per), :],
                send_sem=send_sems.at[h],
                recv_sem=recv_sems.at[h],
                device_id=(right,),
                device_id_type=pl.DeviceIdType.MESH,
            )
            rdma.start()
            rdma.wait()

        y = jnp.dot(comm_ref[...], w_ref[...],
                    preferred_element_type=jnp.float32)
        out_ref[...] = jnp.maximum(y, 0.0)

    return pl.pallas_call(
        body,
        out_shape=jax.ShapeDtypeStruct((m_total, n_per), jnp.float32),
        in_specs=[
            pl.BlockSpec(memory_space=pltpu.VMEM),
            pl.BlockSpec(memory_space=pltpu.VMEM),
        ],
        out_specs=pl.BlockSpec(memory_space=pltpu.VMEM),
        scratch_shapes=[
            pltpu.VMEM((m_total, k), jnp.float32),
            pltpu.SemaphoreType.DMA((N_DEV - 1,)),
            pltpu.SemaphoreType.DMA((N_DEV - 1,)),
        ],
        compiler_params=pltpu.CompilerParams(collective_id=0),
    )(x, w_mat)


# device time: 69204 ns/iter; 1.5539x vs baseline; 1.5539x over previous
import jax
import jax.numpy as jnp
from jax import lax
from jax.experimental import pallas as pl
from jax.experimental.pallas import tpu as pltpu

N_DEV = 32
N_FWD = N_DEV // 2
N_BWD = N_DEV - 1 - N_FWD


def kernel(x, w_mat):
    m_per, k = x.shape
    _, n_per = w_mat.shape
    m_total = N_DEV * m_per

    def body(x_ref, w_ref, out_ref, comm_ref,
             send_f, recv_f, send_b, recv_b):
        my = lax.axis_index("i")
        left = (my - 1) % N_DEV
        right = (my + 1) % N_DEV

        barrier_sem = pltpu.get_barrier_semaphore()
        for nbr in (left, right):
            pl.semaphore_signal(
                barrier_sem, inc=1,
                device_id=(nbr,), device_id_type=pl.DeviceIdType.MESH,
            )
        pl.semaphore_wait(barrier_sem, 2)

        comm_ref[pl.ds(my * m_per, m_per), :] = x_ref[...]

        def slot(o):
            return comm_ref.at[pl.ds((o % N_DEV) * m_per, m_per), :]

        def fwd_rdma(s):
            o = (my - s) % N_DEV
            return pltpu.make_async_remote_copy(
                src_ref=slot(o), dst_ref=slot(o),
                send_sem=send_f.at[s], recv_sem=recv_f.at[s],
                device_id=(right,), device_id_type=pl.DeviceIdType.MESH,
            )

        def bwd_rdma(s):
            o = (my + s) % N_DEV
            return pltpu.make_async_remote_copy(
                src_ref=slot(o), dst_ref=slot(o),
                send_sem=send_b.at[s], recv_sem=recv_b.at[s],
                device_id=(left,), device_id_type=pl.DeviceIdType.MESH,
            )

        def fwd_wait(s):
            o = (my - 1 - s) % N_DEV
            return pltpu.make_async_remote_copy(
                src_ref=slot(o), dst_ref=slot(o),
                send_sem=send_f.at[s], recv_sem=recv_f.at[s],
                device_id=(right,), device_id_type=pl.DeviceIdType.MESH,
            )

        def bwd_wait(s):
            o = (my + 1 + s) % N_DEV
            return pltpu.make_async_remote_copy(
                src_ref=slot(o), dst_ref=slot(o),
                send_sem=send_b.at[s], recv_sem=recv_b.at[s],
                device_id=(left,), device_id_type=pl.DeviceIdType.MESH,
            )

        started = []

        r0 = fwd_rdma(0); r0.start(); started.append(r0)
        b0 = bwd_rdma(0); b0.start(); started.append(b0)

        for s in range(1, max(N_FWD, N_BWD)):
            if s < N_FWD:
                fwd_wait(s - 1).wait_recv()
                r = fwd_rdma(s); r.start(); started.append(r)
            if s < N_BWD:
                bwd_wait(s - 1).wait_recv()
                r = bwd_rdma(s); r.start(); started.append(r)

        fwd_wait(N_FWD - 1).wait_recv()
        bwd_wait(N_BWD - 1).wait_recv()

        for r in started:
            r.wait_send()

        y = jnp.dot(comm_ref[...], w_ref[...],
                    preferred_element_type=jnp.float32)
        out_ref[...] = jnp.maximum(y, 0.0)

    return pl.pallas_call(
        body,
        out_shape=jax.ShapeDtypeStruct((m_total, n_per), jnp.float32),
        in_specs=[
            pl.BlockSpec(memory_space=pltpu.VMEM),
            pl.BlockSpec(memory_space=pltpu.VMEM),
        ],
        out_specs=pl.BlockSpec(memory_space=pltpu.VMEM),
        scratch_shapes=[
            pltpu.VMEM((m_total, k), jnp.float32),
            pltpu.SemaphoreType.DMA((N_FWD,)),
            pltpu.SemaphoreType.DMA((N_FWD,)),
            pltpu.SemaphoreType.DMA((N_BWD,)),
            pltpu.SemaphoreType.DMA((N_BWD,)),
        ],
        compiler_params=pltpu.CompilerParams(collective_id=0),
    )(x, w_mat)


# device time: 51339 ns/iter; 2.0946x vs baseline; 1.3480x over previous
import jax
import jax.numpy as jnp
from jax import lax
from jax.experimental import pallas as pl
from jax.experimental.pallas import tpu as pltpu

N_DEV = 32
HALF = N_DEV // 2
STEPS = 8


def kernel(x, w_mat):
    m_per, k = x.shape
    _, n_per = w_mat.shape
    m_total = N_DEV * m_per

    def body(x_ref, w_ref, out_ref, comm_ref,
             r1s, r1r, r2s, r2r, l1s, l1r, l2s, l2r, js, jr):
        my = lax.axis_index("i")
        left = (my - 1) % N_DEV
        right = (my + 1) % N_DEV
        opp = (my + HALF) % N_DEV

        barrier_sem = pltpu.get_barrier_semaphore()
        for nbr in (left, right, opp):
            pl.semaphore_signal(
                barrier_sem, inc=1,
                device_id=(nbr,), device_id_type=pl.DeviceIdType.MESH,
            )
        pl.semaphore_wait(barrier_sem, 3)

        comm_ref[pl.ds(my * m_per, m_per), :] = x_ref[...]

        def slot(o):
            return comm_ref.at[pl.ds((o % N_DEV) * m_per, m_per), :]

        def xfer(o, ssem, rsem, target):
            return pltpu.make_async_remote_copy(
                src_ref=slot(o), dst_ref=slot(o),
                send_sem=ssem, recv_sem=rsem,
                device_id=(target,), device_id_type=pl.DeviceIdType.MESH,
            )

        def R1(s):
            return xfer(my - s, r1s.at[s], r1r.at[s], right)

        def L1(s):
            return xfer(my + s, l1s.at[s], l1r.at[s], left)

        def R2(s):
            return xfer(my + 17 - s, r2s.at[s], r2r.at[s], right)

        def L2(s):
            return xfer(my - 17 + s, l2s.at[s], l2r.at[s], left)

        def R1_in(s):
            return xfer(my - 1 - s, r1s.at[s], r1r.at[s], right)

        def L1_in(s):
            return xfer(my + 1 + s, l1s.at[s], l1r.at[s], left)

        def R2_in(s):
            return xfer(my + 16 - s, r2s.at[s], r2r.at[s], right)

        def L2_in(s):
            return xfer(my - 16 + s, l2s.at[s], l2r.at[s], left)

        started = []

        def start(r):
            r.start()
            started.append(r)

        start(xfer(my, js.at[0], jr.at[0], opp))
        jump_in = xfer(my + HALF, js.at[0], jr.at[0], opp)

        start(R1(0))
        start(L1(0))

        for s in range(1, STEPS):
            R1_in(s - 1).wait_recv()
            start(R1(s))
            L1_in(s - 1).wait_recv()
            start(L1(s))
            if s == 1:
                jump_in.wait_recv()
            else:
                R2_in(s - 1).wait_recv()
                L2_in(s - 1).wait_recv()
            start(R2(s))
            start(L2(s))

        R1_in(STEPS - 1).wait_recv()
        L1_in(STEPS - 1).wait_recv()
        R2_in(STEPS - 1).wait_recv()
        L2_in(STEPS - 1).wait_recv()

        for r in started:
            r.wait_send()

        y = jnp.dot(comm_ref[...], w_ref[...],
                    preferred_element_type=jnp.float32)
        out_ref[...] = jnp.maximum(y, 0.0)

    return pl.pallas_call(
        body,
        out_shape=jax.ShapeDtypeStruct((m_total, n_per), jnp.float32),
        in_specs=[
            pl.BlockSpec(memory_space=pltpu.VMEM),
            pl.BlockSpec(memory_space=pltpu.VMEM),
        ],
        out_specs=pl.BlockSpec(memory_space=pltpu.VMEM),
        scratch_shapes=[
            pltpu.VMEM((m_total, k), jnp.float32),
            pltpu.SemaphoreType.DMA((STEPS,)),
            pltpu.SemaphoreType.DMA((STEPS,)),
            pltpu.SemaphoreType.DMA((STEPS,)),
            pltpu.SemaphoreType.DMA((STEPS,)),
            pltpu.SemaphoreType.DMA((STEPS,)),
            pltpu.SemaphoreType.DMA((STEPS,)),
            pltpu.SemaphoreType.DMA((STEPS,)),
            pltpu.SemaphoreType.DMA((STEPS,)),
            pltpu.SemaphoreType.DMA((1,)),
            pltpu.SemaphoreType.DMA((1,)),
        ],
        compiler_params=pltpu.CompilerParams(collective_id=0),
    )(x, w_mat)


# device time: 49895 ns/iter; 2.1552x vs baseline; 1.0289x over previous
import jax
import jax.numpy as jnp
from jax import lax
from jax.experimental import pallas as pl
from jax.experimental.pallas import tpu as pltpu

N_DEV = 32
INJ = (0, 8, 16, 24)
RLEN = {0: 4, 8: 3, 16: 4, 24: 3}
LLEN = {0: 4, 8: 3, 16: 4, 24: 3}
MAX_S = 4


def kernel(x, w_mat):
    m_per, k = x.shape
    _, n_per = w_mat.shape
    m_total = N_DEV * m_per

    def body(x_ref, w_ref, out_ref, comm_ref,
             rsend, rrecv, lsend, lrecv, jsend, jrecv):
        my = lax.axis_index("i")
        left = (my - 1) % N_DEV
        right = (my + 1) % N_DEV

        barrier_sem = pltpu.get_barrier_semaphore()
        peers = (left, right, (my + 8) % N_DEV, (my - 8) % N_DEV,
                 (my + 16) % N_DEV)
        for nbr in peers:
            pl.semaphore_signal(
                barrier_sem, inc=1,
                device_id=(nbr,), device_id_type=pl.DeviceIdType.MESH,
            )
        pl.semaphore_wait(barrier_sem, len(peers))

        comm_ref[pl.ds(my * m_per, m_per), :] = x_ref[...]

        def slot(o):
            return comm_ref.at[pl.ds((o % N_DEV) * m_per, m_per), :]

        def xfer(o, ssem, rsem, target):
            return pltpu.make_async_remote_copy(
                src_ref=slot(o), dst_ref=slot(o),
                send_sem=ssem, recv_sem=rsem,
                device_id=(target,), device_id_type=pl.DeviceIdType.MESH,
            )

        def R(ji, s):
            return xfer(my - INJ[ji] - s + 1, rsend.at[ji, s - 1],
                        rrecv.at[ji, s - 1], right)

        def R_in(ji, s):
            return xfer(my - INJ[ji] - s, rsend.at[ji, s - 1],
                        rrecv.at[ji, s - 1], right)

        def L(ji, s):
            return xfer(my - INJ[ji] + s - 1, lsend.at[ji, s - 1],
                        lrecv.at[ji, s - 1], left)

        def L_in(ji, s):
            return xfer(my - INJ[ji] + s, lsend.at[ji, s - 1],
                        lrecv.at[ji, s - 1], left)

        def J(ji):
            return xfer(my, jsend.at[ji - 1], jrecv.at[ji - 1],
                        (my + INJ[ji]) % N_DEV)

        def J_in(ji):
            return xfer(my - INJ[ji], jsend.at[ji - 1], jrecv.at[ji - 1],
                        (my + INJ[ji]) % N_DEV)

        started = []

        def start(r):
            r.start()
            started.append(r)

        for ji in (1, 2, 3):
            start(J(ji))

        start(R(0, 1))
        start(L(0, 1))

        for s in range(1, MAX_S + 1):
            for ji in range(4):
                if s == 1:
                    if ji == 0:
                        continue
                    J_in(ji).wait_recv()
                else:
                    if s <= RLEN[INJ[ji]]:
                        R_in(ji, s - 1).wait_recv()
                    if s <= LLEN[INJ[ji]]:
                        L_in(ji, s - 1).wait_recv()
                if s <= RLEN[INJ[ji]]:
                    start(R(ji, s))
                if s <= LLEN[INJ[ji]]:
                    start(L(ji, s))

        for ji in range(4):
            R_in(ji, RLEN[INJ[ji]]).wait_recv()
            L_in(ji, LLEN[INJ[ji]]).wait_recv()

        for r in started:
            r.wait_send()

        y = jnp.dot(comm_ref[...], w_ref[...],
                    preferred_element_type=jnp.float32)
        out_ref[...] = jnp.maximum(y, 0.0)

    return pl.pallas_call(
        body,
        out_shape=jax.ShapeDtypeStruct((m_total, n_per), jnp.float32),
        in_specs=[
            pl.BlockSpec(memory_space=pltpu.VMEM),
            pl.BlockSpec(memory_space=pltpu.VMEM),
        ],
        out_specs=pl.BlockSpec(memory_space=pltpu.VMEM),
        scratch_shapes=[
            pltpu.VMEM((m_total, k), jnp.float32),
            pltpu.SemaphoreType.DMA((4, MAX_S)),
            pltpu.SemaphoreType.DMA((4, MAX_S)),
            pltpu.SemaphoreType.DMA((4, MAX_S)),
            pltpu.SemaphoreType.DMA((4, MAX_S)),
            pltpu.SemaphoreType.DMA((3,)),
            pltpu.SemaphoreType.DMA((3,)),
        ],
        compiler_params=pltpu.CompilerParams(collective_id=0),
    )(x, w_mat)
